# baseline (speedup 1.0000x reference)
import functools

import jax
import jax.numpy as jnp
from jax import lax
from jax.experimental import pallas as pl
from jax.experimental.pallas import tpu as pltpu
from jax.experimental.pallas import tpu_sc as plsc

_LANES = 16


def _chunk_schedule(rows_per_worker):
    if rows_per_worker < 96:
        assert rows_per_worker % 16 == 0
        sizes = [16, 16, 16]
        chunks = [(o, 16, o // 16 % 3) for o in range(0, rows_per_worker, 16)]
        return chunks, sizes
    sizes = [8, 8, 32, 32, 32]
    mid_rows = rows_per_worker - 32
    assert mid_rows % 32 == 0
    chunks = [(0, 8, 0), (8, 8, 1)]
    off = 16
    for i in range(mid_rows // 32):
        chunks.append((off, 32, 2 + i % 3))
        off += 32
    chunks += [(off, 8, 0), (off + 8, 8, 1)]
    return chunks, sizes


@functools.lru_cache(maxsize=None)
def _make_scale_kernel(rows: int, dim: int, scale: float):
    info = plsc.get_sparse_core_info()
    num_workers = info.num_cores * info.num_subcores
    assert rows % num_workers == 0
    rows_per_worker = rows // num_workers
    chunks, slot_sizes = _chunk_schedule(rows_per_worker)
    n_slots = len(slot_sizes)
    depth = 2

    mesh = plsc.VectorSubcoreMesh(core_axis_name="c", subcore_axis_name="s")

    scratch = [pltpu.VMEM((sz, dim), jnp.float32) for sz in slot_sizes]
    scratch += [pltpu.SemaphoreType.DMA] * (2 * n_slots)

    @functools.partial(
        pl.kernel,
        mesh=mesh,
        out_type=jax.ShapeDtypeStruct((rows, dim), jnp.float32),
        scratch_types=scratch,
        compiler_params=pltpu.CompilerParams(
            use_tc_tiling_on_sc=True,
            disable_bounds_checks=True,
            skip_device_barrier=True,
        ),
    )
    def scale_kernel(emb_hbm, out_hbm, *refs):
        bufs = refs[:n_slots]
        sin = refs[n_slots:2 * n_slots]
        sout = refs[2 * n_slots:3 * n_slots]
        wid = lax.axis_index("s") * info.num_cores + lax.axis_index("c")
        base = wid * rows_per_worker

        in_copies = {}
        out_copies = {}

        def start_in(ci):
            off, nr, slot = chunks[ci]
            if slot in out_copies:
                out_copies.pop(slot).wait()
            in_copies[slot] = pltpu.async_copy(
                emb_hbm.at[pl.ds(base + off, nr)], bufs[slot], sin[slot])

        for p in range(min(depth, len(chunks))):
            start_in(p)
        for ci, (off, nr, slot) in enumerate(chunks):
            if ci + depth < len(chunks):
                start_in(ci + depth)
            in_copies.pop(slot).wait()
            buf = bufs[slot]

            def row_body(r, _):
                @plsc.parallel_loop(0, dim, step=_LANES, unroll=16)
                def _scale(i):
                    buf[r, pl.ds(i, _LANES)] = buf[r, pl.ds(i, _LANES)] * scale

                return 0

            lax.fori_loop(0, nr, row_body, 0)

            out_copies[slot] = pltpu.async_copy(
                buf, out_hbm.at[pl.ds(base + off, nr)], sout[slot])
        for copy in out_copies.values():
            copy.wait()

    return scale_kernel


def kernel(x, emb_weight):
    seq_len = x.shape[1]
    dim = emb_weight.shape[1]
    scale = dim ** -0.5
    return _make_scale_kernel(seq_len, dim, scale)(emb_weight[:seq_len])


# speedup vs baseline: 1.0112x; 1.0112x over previous
import functools

import jax
import jax.numpy as jnp
from jax import lax
from jax.experimental import pallas as pl
from jax.experimental.pallas import tpu as pltpu
from jax.experimental.pallas import tpu_sc as plsc

_LANES = 16


def _chunk_schedule(rows_per_worker):
    if rows_per_worker < 96:
        assert rows_per_worker % 16 == 0
        sizes = [16, 16, 16]
        chunks = [(o, 16, o // 16 % 3) for o in range(0, rows_per_worker, 16)]
        return chunks, sizes
    sizes = [8, 8, 32, 32, 32]
    mid_rows = rows_per_worker - 32
    assert mid_rows % 32 == 0
    chunks = [(0, 8, 0), (8, 8, 1)]
    off = 16
    for i in range(mid_rows // 32):
        chunks.append((off, 32, 2 + i % 3))
        off += 32
    chunks += [(off, 8, 0), (off + 8, 8, 1)]
    return chunks, sizes


@functools.lru_cache(maxsize=None)
def _make_scale_kernel(rows: int, dim: int, scale: float):
    info = plsc.get_sparse_core_info()
    num_workers = info.num_cores * info.num_subcores
    assert rows % num_workers == 0
    rows_per_worker = rows // num_workers
    chunks, slot_sizes = _chunk_schedule(rows_per_worker)
    n_slots = len(slot_sizes)
    depth = 2

    mesh = plsc.VectorSubcoreMesh(core_axis_name="c", subcore_axis_name="s")

    scratch = [pltpu.VMEM((sz, dim), jnp.float32) for sz in slot_sizes]
    scratch += [pltpu.SemaphoreType.DMA] * (2 * n_slots)

    @functools.partial(
        pl.kernel,
        mesh=mesh,
        out_type=jax.ShapeDtypeStruct((rows, dim), jnp.float32),
        scratch_types=scratch,
        compiler_params=pltpu.CompilerParams(
            use_tc_tiling_on_sc=True,
            disable_bounds_checks=True,
            skip_device_barrier=True,
        ),
    )
    def scale_kernel(emb_hbm, out_hbm, *refs):
        bufs = refs[:n_slots]
        sin = refs[n_slots:2 * n_slots]
        sout = refs[2 * n_slots:3 * n_slots]
        wid = lax.axis_index("c") * info.num_subcores + lax.axis_index("s")
        base = wid * rows_per_worker

        in_copies = {}
        out_copies = {}

        def start_in(ci):
            off, nr, slot = chunks[ci]
            if slot in out_copies:
                out_copies.pop(slot).wait()
            in_copies[slot] = pltpu.async_copy(
                emb_hbm.at[pl.ds(base + off, nr)], bufs[slot], sin[slot])

        for p in range(min(depth, len(chunks))):
            start_in(p)
        for ci, (off, nr, slot) in enumerate(chunks):
            if ci + depth < len(chunks):
                start_in(ci + depth)
            in_copies.pop(slot).wait()
            buf = bufs[slot]

            def row_body(r, _):
                @plsc.parallel_loop(0, dim, step=_LANES, unroll=8)
                def _scale(i):
                    buf[r, pl.ds(i, _LANES)] = buf[r, pl.ds(i, _LANES)] * scale

                return 0

            lax.fori_loop(0, nr, row_body, 0)

            out_copies[slot] = pltpu.async_copy(
                buf, out_hbm.at[pl.ds(base + off, nr)], sout[slot])
        for copy in out_copies.values():
            copy.wait()

    return scale_kernel


def kernel(x, emb_weight):
    seq_len = x.shape[1]
    dim = emb_weight.shape[1]
    scale = dim ** -0.5
    return _make_scale_kernel(seq_len, dim, scale)(emb_weight[:seq_len])


# speedup vs baseline: 1.0215x; 1.0103x over previous
import functools

import jax
import jax.numpy as jnp
from jax import lax
from jax.experimental import pallas as pl
from jax.experimental.pallas import tpu as pltpu
from jax.experimental.pallas import tpu_sc as plsc

_LANES = 16


def _chunk_schedule(rows_per_worker):
    if rows_per_worker < 96:
        assert rows_per_worker % 16 == 0
        sizes = [16, 16, 16]
        chunks = [(o, 16, o // 16 % 3) for o in range(0, rows_per_worker, 16)]
        return chunks, sizes
    sizes = [8, 8, 16, 16, 16, 16, 16, 16]
    mid_rows = rows_per_worker - 32
    assert mid_rows % 16 == 0
    chunks = [(0, 8, 0), (8, 8, 1)]
    off = 16
    for i in range(mid_rows // 16):
        chunks.append((off, 16, 2 + i % 6))
        off += 16
    chunks += [(off, 8, 0), (off + 8, 8, 1)]
    return chunks, sizes


@functools.lru_cache(maxsize=None)
def _make_scale_kernel(rows: int, dim: int, scale: float):
    info = plsc.get_sparse_core_info()
    num_workers = info.num_cores * info.num_subcores
    assert rows % num_workers == 0
    rows_per_worker = rows // num_workers
    chunks, slot_sizes = _chunk_schedule(rows_per_worker)
    n_slots = len(slot_sizes)
    depth = 5

    mesh = plsc.VectorSubcoreMesh(core_axis_name="c", subcore_axis_name="s")

    scratch = [pltpu.VMEM((sz, dim), jnp.float32) for sz in slot_sizes]
    scratch += [pltpu.SemaphoreType.DMA] * (2 * n_slots)

    @functools.partial(
        pl.kernel,
        mesh=mesh,
        out_type=jax.ShapeDtypeStruct((rows, dim), jnp.float32),
        scratch_types=scratch,
        compiler_params=pltpu.CompilerParams(
            use_tc_tiling_on_sc=True,
            disable_bounds_checks=True,
            skip_device_barrier=True,
        ),
    )
    def scale_kernel(emb_hbm, out_hbm, *refs):
        bufs = refs[:n_slots]
        sin = refs[n_slots:2 * n_slots]
        sout = refs[2 * n_slots:3 * n_slots]
        wid = lax.axis_index("c") * info.num_subcores + lax.axis_index("s")
        base = wid * rows_per_worker

        in_copies = {}
        out_copies = {}

        def start_in(ci):
            off, nr, slot = chunks[ci]
            if slot in out_copies:
                out_copies.pop(slot).wait()
            in_copies[slot] = pltpu.async_copy(
                emb_hbm.at[pl.ds(base + off, nr)], bufs[slot], sin[slot])

        for p in range(min(depth, len(chunks))):
            start_in(p)
        for ci, (off, nr, slot) in enumerate(chunks):
            if ci + depth < len(chunks):
                start_in(ci + depth)
            in_copies.pop(slot).wait()
            buf = bufs[slot]

            def row_body(r, _):
                @plsc.parallel_loop(0, dim, step=_LANES, unroll=8)
                def _scale(i):
                    buf[r, pl.ds(i, _LANES)] = buf[r, pl.ds(i, _LANES)] * scale

                return 0

            lax.fori_loop(0, nr, row_body, 0)

            out_copies[slot] = pltpu.async_copy(
                buf, out_hbm.at[pl.ds(base + off, nr)], sout[slot])
        for copy in out_copies.values():
            copy.wait()

    return scale_kernel


def kernel(x, emb_weight):
    seq_len = x.shape[1]
    dim = emb_weight.shape[1]
    scale = dim ** -0.5
    return _make_scale_kernel(seq_len, dim, scale)(emb_weight[:seq_len])
